# baseline (speedup 1.0000x reference)
import functools

import jax
import jax.numpy as jnp
from jax import lax
from jax.experimental import pallas as pl
from jax.experimental.pallas import tpu as pltpu
from jax.experimental.pallas import tpu_sc as plsc

N = 10000
E = 320000
D = 128
H = 128
HGB = 16
NGB = H // HGB
OUT = 16

KB = 80
KBB = 160
NB_A = E // 32 // KB
NB_B = E // 16 // KBB
NDEN = 10240
NAGG = 10000

_mesh = plsc.VectorSubcoreMesh(core_axis_name="c", subcore_axis_name="s")


def _mm_body(x_ref, win_ref, bin_ref, wg_ref, asrc_ref, adst_ref,
             h0_ref, h_ref, as_ref, ad_ref, mx_ref):
    x = x_ref[...]
    h0 = jnp.dot(x, win_ref[...], preferred_element_type=jnp.float32) + bin_ref[...]
    h = jnp.dot(h0, wg_ref[...], preferred_element_type=jnp.float32)
    h0_ref[...] = h0
    h_ref[...] = h
    a_s = jnp.dot(h, asrc_ref[...], preferred_element_type=jnp.float32)
    a_d = jnp.dot(h, adst_ref[...], preferred_element_type=jnp.float32)
    as_ref[...] = a_s
    ad_ref[...] = a_d
    ms = jnp.max(a_s)
    md = jnp.max(a_d)
    cur = jnp.concatenate([jnp.full((1, 128), ms, jnp.float32),
                           jnp.full((1, 128), md, jnp.float32)], axis=1)

    @pl.when(pl.program_id(0) == 0)
    def _():
        mx_ref[...] = cur

    @pl.when(pl.program_id(0) > 0)
    def _():
        mx_ref[...] = jnp.maximum(mx_ref[...], cur)


_RB = 2000


def _mm_call(x, w_in, b_in, w_g, att_src, att_dst):
    grid = (N // _RB,)
    return pl.pallas_call(
        _mm_body,
        grid=grid,
        in_specs=[
            pl.BlockSpec((_RB, D), lambda i: (i, 0)),
            pl.BlockSpec((D, H), lambda i: (0, 0)),
            pl.BlockSpec((1, H), lambda i: (0, 0)),
            pl.BlockSpec((H, H), lambda i: (0, 0)),
            pl.BlockSpec((H, 1), lambda i: (0, 0)),
            pl.BlockSpec((H, 1), lambda i: (0, 0)),
        ],
        out_specs=[
            pl.BlockSpec((_RB, H), lambda i: (i, 0)),
            pl.BlockSpec((_RB, H), lambda i: (i, 0)),
            pl.BlockSpec((_RB, 1), lambda i: (i, 0)),
            pl.BlockSpec((_RB, 1), lambda i: (i, 0)),
            pl.BlockSpec((1, 256), lambda i: (0, 0)),
        ],
        out_shape=[
            jax.ShapeDtypeStruct((N, H), jnp.float32),
            jax.ShapeDtypeStruct((N, H), jnp.float32),
            jax.ShapeDtypeStruct((N, 1), jnp.float32),
            jax.ShapeDtypeStruct((N, 1), jnp.float32),
            jax.ShapeDtypeStruct((1, 256), jnp.float32),
        ],
        compiler_params=pltpu.CompilerParams(
            dimension_semantics=("arbitrary",)),
    )(x, w_in, b_in, w_g, att_src, att_dst)


@functools.partial(
    pl.kernel,
    mesh=_mesh,
    out_type=jax.ShapeDtypeStruct((2, NDEN), jnp.float32),
    compiler_params=pltpu.CompilerParams(needs_layout_passes=False),
    scratch_types=[
        pltpu.VMEM((N,), jnp.float32),
        pltpu.VMEM((N,), jnp.float32),
        pltpu.VMEM((NB_A, KB), jnp.int32),
        pltpu.VMEM((NB_A, KB), jnp.int32),
        pltpu.VMEM((KB,), jnp.float32),
        pltpu.VMEM((16,), jnp.float32),
        pltpu.VMEM_SHARED((NDEN,), jnp.float32),
    ],
)
def _sc_den(as_hbm, ad_hbm, src_hbm, dst_hbm, shift_hbm, z640_hbm, denp_hbm,
            as_v, ad_v, src_v, dst_v, ex_v, shift_v, den_sh):
    c = lax.axis_index("c")
    s = lax.axis_index("s")
    wid = c * 16 + s
    off = pl.multiple_of(s * 640, 640)
    pltpu.sync_copy(z640_hbm, den_sh.at[pl.ds(off, 640)])
    pltpu.sync_copy(as_hbm, as_v)
    pltpu.sync_copy(ad_hbm, ad_v)
    pltpu.sync_copy(src_hbm.at[wid], src_v)
    pltpu.sync_copy(dst_hbm.at[wid], dst_v)
    pltpu.sync_copy(shift_hbm, shift_v)
    plsc.subcore_barrier()
    shift = shift_v[...]

    def body(b, carry):
        for cc in range(KB // 16):
            sv = src_v[b, pl.ds(cc * 16, 16)]
            dv = dst_v[b, pl.ds(cc * 16, 16)]
            ls = plsc.load_gather(as_v, [sv])
            ld = plsc.load_gather(ad_v, [dv])
            e = ls + ld
            e = jnp.where(e > 0.0, e, 0.2 * e)
            ex_v[pl.ds(cc * 16, 16)] = jnp.exp(e - shift)
        pltpu.sync_copy(ex_v, den_sh.at[dst_v.at[b]], add=True)
        return carry

    lax.fori_loop(0, NB_A, body, 0)
    plsc.subcore_barrier()
    pltpu.sync_copy(den_sh.at[pl.ds(off, 640)],
                    denp_hbm.at[c, pl.ds(off, 640)])


@functools.partial(
    pl.kernel,
    mesh=_mesh,
    out_type=jax.ShapeDtypeStruct((NGB, NAGG, HGB), jnp.float32),
    compiler_params=pltpu.CompilerParams(
        needs_layout_passes=False, use_tc_tiling_on_sc=False),
    scratch_types=[
        pltpu.VMEM((N,), jnp.float32),
        pltpu.VMEM((N,), jnp.float32),
        pltpu.VMEM((2, NDEN), jnp.float32),
        pltpu.VMEM((NDEN,), jnp.float32),
        pltpu.VMEM((NB_B, KBB), jnp.int32),
        pltpu.VMEM((NB_B, KBB), jnp.int32),
        pltpu.VMEM((NB_B, KBB), jnp.float32),
        pltpu.VMEM((2, KBB, HGB), jnp.float32),
        pltpu.VMEM((2, KBB, HGB), jnp.float32),
        pltpu.VMEM((16,), jnp.float32),
        pltpu.VMEM_SHARED((NAGG, HGB), jnp.float32),
        pltpu.SemaphoreType.DMA,
        pltpu.SemaphoreType.DMA,
    ],
)
def _sc_agg(as_hbm, ad_hbm, denp_hbm, src_hbm, dst_hbm, shift_hbm,
            zrows_hbm, hq_hbm, aggq_hbm,
            as_v, ad_v, denp_v, den_v, src_v, dst_v, alpha_v, grows_v,
            srows_v, shift_v, agg_sh, gsem, ssem):
    c = lax.axis_index("c")
    s = lax.axis_index("s")
    roff = pl.multiple_of(s * 624, 8)
    last = s == 15
    pltpu.sync_copy(as_hbm, as_v)
    pltpu.sync_copy(ad_hbm, ad_v)
    pltpu.sync_copy(denp_hbm, denp_v)
    pltpu.sync_copy(src_hbm.at[s], src_v)
    pltpu.sync_copy(dst_hbm.at[s], dst_v)
    pltpu.sync_copy(shift_hbm, shift_v)

    def dsum(i, carry):
        o = pl.multiple_of(i * 16, 16)
        den_v[pl.ds(o, 16)] = denp_v[0, pl.ds(o, 16)] + denp_v[1, pl.ds(o, 16)]
        return carry

    lax.fori_loop(0, NDEN // 16, dsum, 0)
    shift = shift_v[...]
    nofs = jnp.full((16,), N, jnp.int32)
    ofs0 = jnp.full((16,), 4 * N, jnp.int32) * c

    for q in range(4):
        @pl.when(jnp.logical_not(last))
        def _(q=q):
            pltpu.sync_copy(zrows_hbm.at[pl.ds(0, 624)],
                            agg_sh.at[pl.ds(roff, 624)])

        @pl.when(last)
        def _(q=q):
            pltpu.sync_copy(zrows_hbm, agg_sh.at[pl.ds(9360, 640)])

        def prep(b, carry, q=q):
            for cc in range(KBB // 16):
                sl = pl.ds(cc * 16, 16)
                if q == 0:
                    sv = src_v[b, sl]
                    dv = dst_v[b, sl]
                    ls = plsc.load_gather(as_v, [sv])
                    ld = plsc.load_gather(ad_v, [dv])
                    e = ls + ld
                    e = jnp.where(e > 0.0, e, 0.2 * e)
                    ex = jnp.exp(e - shift)
                    dd = plsc.load_gather(den_v, [dv])
                    alpha_v[b, sl] = ex / dd
                    src_v[b, sl] = sv + ofs0
                else:
                    src_v[b, sl] = src_v[b, sl] + nofs
            return carry

        lax.fori_loop(0, NB_B, prep, 0)
        plsc.subcore_barrier()
        pltpu.async_copy(hq_hbm.at[src_v.at[0]], grows_v.at[0], gsem)

        def body(b, carry):
            bi = lax.rem(b, 2)
            ni = lax.rem(b + 1, 2)

            @pl.when(b >= 1)
            def _():
                pltpu.make_async_copy(
                    srows_v.at[0], agg_sh.at[dst_v.at[0]], ssem).wait()

            @pl.when(b + 1 < NB_B)
            def _():
                pltpu.async_copy(
                    hq_hbm.at[src_v.at[b + 1]], grows_v.at[ni], gsem)

            pltpu.make_async_copy(
                hq_hbm.at[src_v.at[b]], grows_v.at[bi], gsem).wait()

            def scale(k8, carry2):
                for dk in range(8):
                    k = k8 * 8 + dk
                    av = plsc.load_gather(
                        alpha_v, [jnp.full((16,), b, jnp.int32),
                                  jnp.full((16,), k, jnp.int32)])
                    srows_v[bi, k, pl.ds(0, HGB)] = (
                        grows_v[bi, k, pl.ds(0, HGB)] * av)
                return carry2

            lax.fori_loop(0, KBB // 8, scale, 0)
            pltpu.async_copy(srows_v.at[bi], agg_sh.at[dst_v.at[b]], ssem,
                             add=True)
            return carry

        lax.fori_loop(0, NB_B, body, 0)
        pltpu.make_async_copy(
            srows_v.at[0], agg_sh.at[dst_v.at[0]], ssem).wait()
        plsc.subcore_barrier()

        @pl.when(jnp.logical_not(last))
        def _(q=q):
            pltpu.sync_copy(agg_sh.at[pl.ds(roff, 624)],
                            aggq_hbm.at[4 * c + q, pl.ds(roff, 624)])

        @pl.when(last)
        def _(q=q):
            pltpu.sync_copy(agg_sh.at[pl.ds(9360, 640)],
                            aggq_hbm.at[4 * c + q, pl.ds(9360, 640)])


def _out_body(h0_ref, aggq_ref, bias_ref, wout_ref, bout_ref, o_ref):
    agg = jnp.concatenate(
        [aggq_ref[g] for g in range(NGB)], axis=1) + bias_ref[...]
    h1 = jnp.maximum(agg, 0.0)
    h2 = h0_ref[...] + h1
    lg = jnp.dot(h2, wout_ref[...], preferred_element_type=jnp.float32) + bout_ref[...]
    m = jnp.max(lg, axis=1, keepdims=True)
    ex = jnp.exp(lg - m)
    lse = jnp.log(jnp.sum(ex, axis=1, keepdims=True))
    o_ref[...] = lg - m - lse


def _out_call(h0, aggq, bias_g, w_out, b_out):
    grid = (N // _RB,)
    return pl.pallas_call(
        _out_body,
        grid=grid,
        in_specs=[
            pl.BlockSpec((_RB, H), lambda i: (i, 0)),
            pl.BlockSpec((NGB, _RB, HGB), lambda i: (0, i, 0)),
            pl.BlockSpec((1, H), lambda i: (0, 0)),
            pl.BlockSpec((H, OUT), lambda i: (0, 0)),
            pl.BlockSpec((1, OUT), lambda i: (0, 0)),
        ],
        out_specs=pl.BlockSpec((_RB, OUT), lambda i: (i, 0)),
        out_shape=jax.ShapeDtypeStruct((N, OUT), jnp.float32),
    )(h0, aggq, bias_g, w_out, b_out)


def kernel(x, edge_index, W_in, b_in, W_g, att_src, att_dst, bias_g, W_out, b_out):
    src_a = edge_index[0].reshape(32, NB_A, KB)
    dst_a = edge_index[1].reshape(32, NB_A, KB)
    src_b = edge_index[0].reshape(16, NB_B, KBB)
    dst_b = edge_index[1].reshape(16, NB_B, KBB)
    h0, h, a_s, a_d, mx = _mm_call(
        x, W_in, b_in.reshape(1, H), W_g,
        att_src.reshape(H, 1), att_dst.reshape(H, 1))
    a_s = a_s.reshape(N)
    a_d = a_d.reshape(N)
    hq = h.reshape(N, NGB, HGB).transpose(1, 0, 2).reshape(NGB * N, HGB)
    shift = jnp.maximum(mx[0, 0] + mx[0, 128], 0.0)
    shift_v = jnp.full((16,), shift, jnp.float32)
    z640 = jnp.zeros((640,), jnp.float32)
    zrows = jnp.zeros((640, HGB), jnp.float32)
    denp = _sc_den(a_s, a_d, src_a, dst_a, shift_v, z640)
    aggq = _sc_agg(a_s, a_d, denp, src_b, dst_b, shift_v, zrows, hq)
    return _out_call(h0, aggq, bias_g.reshape(1, H), W_out, b_out.reshape(1, OUT))


# speedup vs baseline: 1.0081x; 1.0081x over previous
import functools

import jax
import jax.numpy as jnp
from jax import lax
from jax.experimental import pallas as pl
from jax.experimental.pallas import tpu as pltpu
from jax.experimental.pallas import tpu_sc as plsc

N = 10000
E = 320000
D = 128
H = 128
HGB = 16
NGB = H // HGB
OUT = 16

KB = 80
KBB = 160
NB_A = E // 32 // KB
NB_B = E // 16 // KBB
NDEN = 10240
NAGG = 10000

_mesh = plsc.VectorSubcoreMesh(core_axis_name="c", subcore_axis_name="s")


def _mm_body(x_ref, win_ref, bin_ref, wg_ref, asrc_ref, adst_ref,
             h0_ref, h_ref, as_ref, ad_ref, mx_ref):
    x = x_ref[...]
    h0 = jnp.dot(x, win_ref[...], preferred_element_type=jnp.float32) + bin_ref[...]
    h = jnp.dot(h0, wg_ref[...], preferred_element_type=jnp.float32)
    h0_ref[...] = h0
    h_ref[...] = h
    a_s = jnp.dot(h, asrc_ref[...], preferred_element_type=jnp.float32)
    a_d = jnp.dot(h, adst_ref[...], preferred_element_type=jnp.float32)
    as_ref[...] = a_s
    ad_ref[...] = a_d
    ms = jnp.max(a_s)
    md = jnp.max(a_d)
    cur = jnp.concatenate([jnp.full((1, 128), ms, jnp.float32),
                           jnp.full((1, 128), md, jnp.float32)], axis=1)

    @pl.when(pl.program_id(0) == 0)
    def _():
        mx_ref[...] = cur

    @pl.when(pl.program_id(0) > 0)
    def _():
        mx_ref[...] = jnp.maximum(mx_ref[...], cur)


_RB = 2000


def _mm_call(x, w_in, b_in, w_g, att_src, att_dst):
    grid = (N // _RB,)
    return pl.pallas_call(
        _mm_body,
        grid=grid,
        in_specs=[
            pl.BlockSpec((_RB, D), lambda i: (i, 0)),
            pl.BlockSpec((D, H), lambda i: (0, 0)),
            pl.BlockSpec((1, H), lambda i: (0, 0)),
            pl.BlockSpec((H, H), lambda i: (0, 0)),
            pl.BlockSpec((H, 1), lambda i: (0, 0)),
            pl.BlockSpec((H, 1), lambda i: (0, 0)),
        ],
        out_specs=[
            pl.BlockSpec((_RB, H), lambda i: (i, 0)),
            pl.BlockSpec((_RB, H), lambda i: (i, 0)),
            pl.BlockSpec((_RB, 1), lambda i: (i, 0)),
            pl.BlockSpec((_RB, 1), lambda i: (i, 0)),
            pl.BlockSpec((1, 256), lambda i: (0, 0)),
        ],
        out_shape=[
            jax.ShapeDtypeStruct((N, H), jnp.float32),
            jax.ShapeDtypeStruct((N, H), jnp.float32),
            jax.ShapeDtypeStruct((N, 1), jnp.float32),
            jax.ShapeDtypeStruct((N, 1), jnp.float32),
            jax.ShapeDtypeStruct((1, 256), jnp.float32),
        ],
        compiler_params=pltpu.CompilerParams(
            dimension_semantics=("arbitrary",)),
    )(x, w_in, b_in, w_g, att_src, att_dst)


@functools.partial(
    pl.kernel,
    mesh=_mesh,
    out_type=jax.ShapeDtypeStruct((2, NDEN), jnp.float32),
    compiler_params=pltpu.CompilerParams(needs_layout_passes=False),
    scratch_types=[
        pltpu.VMEM((N,), jnp.float32),
        pltpu.VMEM((N,), jnp.float32),
        pltpu.VMEM((NB_A, KB), jnp.int32),
        pltpu.VMEM((NB_A, KB), jnp.int32),
        pltpu.VMEM((KB,), jnp.float32),
        pltpu.VMEM((16,), jnp.float32),
        pltpu.VMEM_SHARED((NDEN,), jnp.float32),
    ],
)
def _sc_den(as_hbm, ad_hbm, src_hbm, dst_hbm, shift_hbm, z640_hbm, denp_hbm,
            as_v, ad_v, src_v, dst_v, ex_v, shift_v, den_sh):
    c = lax.axis_index("c")
    s = lax.axis_index("s")
    wid = c * 16 + s
    off = pl.multiple_of(s * 640, 640)
    pltpu.sync_copy(z640_hbm, den_sh.at[pl.ds(off, 640)])
    pltpu.sync_copy(as_hbm, as_v)
    pltpu.sync_copy(ad_hbm, ad_v)
    pltpu.sync_copy(src_hbm.at[wid], src_v)
    pltpu.sync_copy(dst_hbm.at[wid], dst_v)
    pltpu.sync_copy(shift_hbm, shift_v)
    plsc.subcore_barrier()
    shift = shift_v[...]

    def body(b, carry):
        for cc in range(KB // 16):
            sv = src_v[b, pl.ds(cc * 16, 16)]
            dv = dst_v[b, pl.ds(cc * 16, 16)]
            ls = plsc.load_gather(as_v, [sv])
            ld = plsc.load_gather(ad_v, [dv])
            e = ls + ld
            e = jnp.where(e > 0.0, e, 0.2 * e)
            ex_v[pl.ds(cc * 16, 16)] = jnp.exp(e - shift)
        pltpu.sync_copy(ex_v, den_sh.at[dst_v.at[b]], add=True)
        return carry

    lax.fori_loop(0, NB_A, body, 0)
    plsc.subcore_barrier()
    pltpu.sync_copy(den_sh.at[pl.ds(off, 640)],
                    denp_hbm.at[c, pl.ds(off, 640)])


@functools.partial(
    pl.kernel,
    mesh=_mesh,
    out_type=jax.ShapeDtypeStruct((NGB, NAGG, HGB), jnp.float32),
    compiler_params=pltpu.CompilerParams(
        needs_layout_passes=False, use_tc_tiling_on_sc=False),
    scratch_types=[
        pltpu.VMEM((N,), jnp.float32),
        pltpu.VMEM((N,), jnp.float32),
        pltpu.VMEM((2, NDEN), jnp.float32),
        pltpu.VMEM((NDEN,), jnp.float32),
        pltpu.VMEM((NB_B, KBB), jnp.int32),
        pltpu.VMEM((NB_B, KBB), jnp.int32),
        pltpu.VMEM((NB_B * KBB,), jnp.float32),
        pltpu.VMEM((2, KBB, HGB), jnp.float32),
        pltpu.VMEM((2, KBB, HGB), jnp.float32),
        pltpu.VMEM((16,), jnp.float32),
        pltpu.VMEM_SHARED((NAGG, HGB), jnp.float32),
        pltpu.SemaphoreType.DMA,
        pltpu.SemaphoreType.DMA,
    ],
)
def _sc_agg(as_hbm, ad_hbm, denp_hbm, src_hbm, dst_hbm, shift_hbm,
            zrows_hbm, hq_hbm, aggq_hbm,
            as_v, ad_v, denp_v, den_v, src_v, dst_v, alpha_v, grows_v,
            srows_v, shift_v, agg_sh, gsem, ssem):
    c = lax.axis_index("c")
    s = lax.axis_index("s")
    roff = pl.multiple_of(s * 624, 8)
    last = s == 15
    pltpu.sync_copy(as_hbm, as_v)
    pltpu.sync_copy(ad_hbm, ad_v)
    pltpu.sync_copy(denp_hbm, denp_v)
    pltpu.sync_copy(src_hbm.at[s], src_v)
    pltpu.sync_copy(dst_hbm.at[s], dst_v)
    pltpu.sync_copy(shift_hbm, shift_v)

    def dsum(i, carry):
        o = pl.multiple_of(i * 16, 16)
        den_v[pl.ds(o, 16)] = denp_v[0, pl.ds(o, 16)] + denp_v[1, pl.ds(o, 16)]
        return carry

    lax.fori_loop(0, NDEN // 16, dsum, 0)
    shift = shift_v[...]
    nofs = jnp.full((16,), N, jnp.int32)
    ofs0 = jnp.full((16,), 4 * N, jnp.int32) * c

    for q in range(4):
        @pl.when(jnp.logical_not(last))
        def _(q=q):
            pltpu.sync_copy(zrows_hbm.at[pl.ds(0, 624)],
                            agg_sh.at[pl.ds(roff, 624)])

        @pl.when(last)
        def _(q=q):
            pltpu.sync_copy(zrows_hbm, agg_sh.at[pl.ds(9360, 640)])

        def prep(b, carry, q=q):
            for cc in range(KBB // 16):
                sl = pl.ds(cc * 16, 16)
                if q == 0:
                    sv = src_v[b, sl]
                    dv = dst_v[b, sl]
                    ls = plsc.load_gather(as_v, [sv])
                    ld = plsc.load_gather(ad_v, [dv])
                    e = ls + ld
                    e = jnp.where(e > 0.0, e, 0.2 * e)
                    ex = jnp.exp(e - shift)
                    dd = plsc.load_gather(den_v, [dv])
                    alpha_v[pl.ds(pl.multiple_of(b * KBB, 16) + cc * 16, 16)] = ex / dd
                    src_v[b, sl] = sv + ofs0
                else:
                    src_v[b, sl] = src_v[b, sl] + nofs
            return carry

        lax.fori_loop(0, NB_B, prep, 0)
        plsc.subcore_barrier()
        pltpu.async_copy(hq_hbm.at[src_v.at[0]], grows_v.at[0], gsem)

        def body(b, carry):
            bi = lax.rem(b, 2)
            ni = lax.rem(b + 1, 2)

            @pl.when(b >= 1)
            def _():
                pltpu.make_async_copy(
                    srows_v.at[0], agg_sh.at[dst_v.at[0]], ssem).wait()

            @pl.when(b + 1 < NB_B)
            def _():
                pltpu.async_copy(
                    hq_hbm.at[src_v.at[b + 1]], grows_v.at[ni], gsem)

            pltpu.make_async_copy(
                hq_hbm.at[src_v.at[b]], grows_v.at[bi], gsem).wait()

            abase = jnp.full((16,), b * KBB, jnp.int32)

            def scale(k16, carry2):
                for dk in range(16):
                    k = k16 * 16 + dk
                    av = plsc.load_gather(alpha_v, [abase + k])
                    srows_v[bi, k, pl.ds(0, HGB)] = (
                        grows_v[bi, k, pl.ds(0, HGB)] * av)
                return carry2

            lax.fori_loop(0, KBB // 16, scale, 0)
            pltpu.async_copy(srows_v.at[bi], agg_sh.at[dst_v.at[b]], ssem,
                             add=True)
            return carry

        lax.fori_loop(0, NB_B, body, 0)
        pltpu.make_async_copy(
            srows_v.at[0], agg_sh.at[dst_v.at[0]], ssem).wait()
        plsc.subcore_barrier()

        @pl.when(jnp.logical_not(last))
        def _(q=q):
            pltpu.sync_copy(agg_sh.at[pl.ds(roff, 624)],
                            aggq_hbm.at[4 * c + q, pl.ds(roff, 624)])

        @pl.when(last)
        def _(q=q):
            pltpu.sync_copy(agg_sh.at[pl.ds(9360, 640)],
                            aggq_hbm.at[4 * c + q, pl.ds(9360, 640)])


def _out_body(h0_ref, aggq_ref, bias_ref, wout_ref, bout_ref, o_ref):
    agg = jnp.concatenate(
        [aggq_ref[g] for g in range(NGB)], axis=1) + bias_ref[...]
    h1 = jnp.maximum(agg, 0.0)
    h2 = h0_ref[...] + h1
    lg = jnp.dot(h2, wout_ref[...], preferred_element_type=jnp.float32) + bout_ref[...]
    m = jnp.max(lg, axis=1, keepdims=True)
    ex = jnp.exp(lg - m)
    lse = jnp.log(jnp.sum(ex, axis=1, keepdims=True))
    o_ref[...] = lg - m - lse


def _out_call(h0, aggq, bias_g, w_out, b_out):
    grid = (N // _RB,)
    return pl.pallas_call(
        _out_body,
        grid=grid,
        in_specs=[
            pl.BlockSpec((_RB, H), lambda i: (i, 0)),
            pl.BlockSpec((NGB, _RB, HGB), lambda i: (0, i, 0)),
            pl.BlockSpec((1, H), lambda i: (0, 0)),
            pl.BlockSpec((H, OUT), lambda i: (0, 0)),
            pl.BlockSpec((1, OUT), lambda i: (0, 0)),
        ],
        out_specs=pl.BlockSpec((_RB, OUT), lambda i: (i, 0)),
        out_shape=jax.ShapeDtypeStruct((N, OUT), jnp.float32),
    )(h0, aggq, bias_g, w_out, b_out)


def kernel(x, edge_index, W_in, b_in, W_g, att_src, att_dst, bias_g, W_out, b_out):
    src_a = edge_index[0].reshape(32, NB_A, KB)
    dst_a = edge_index[1].reshape(32, NB_A, KB)
    src_b = edge_index[0].reshape(16, NB_B, KBB)
    dst_b = edge_index[1].reshape(16, NB_B, KBB)
    h0, h, a_s, a_d, mx = _mm_call(
        x, W_in, b_in.reshape(1, H), W_g,
        att_src.reshape(H, 1), att_dst.reshape(H, 1))
    a_s = a_s.reshape(N)
    a_d = a_d.reshape(N)
    hq = h.reshape(N, NGB, HGB).transpose(1, 0, 2).reshape(NGB * N, HGB)
    shift = jnp.maximum(mx[0, 0] + mx[0, 128], 0.0)
    shift_v = jnp.full((16,), shift, jnp.float32)
    z640 = jnp.zeros((640,), jnp.float32)
    zrows = jnp.zeros((640, HGB), jnp.float32)
    denp = _sc_den(a_s, a_d, src_a, dst_a, shift_v, z640)
    aggq = _sc_agg(a_s, a_d, denp, src_b, dst_b, shift_v, zrows, hq)
    return _out_call(h0, aggq, bias_g.reshape(1, H), W_out, b_out.reshape(1, OUT))


# speedup vs baseline: 1.0149x; 1.0068x over previous
import functools

import jax
import jax.numpy as jnp
from jax import lax
from jax.experimental import pallas as pl
from jax.experimental.pallas import tpu as pltpu
from jax.experimental.pallas import tpu_sc as plsc

N = 10000
E = 320000
D = 128
H = 128
HGB = 16
NGB = H // HGB
OUT = 16

KB = 80
KBB = 160
NB_A = E // 32 // KB
NB_B = E // 16 // KBB
NDEN = 10240
NAGG = 10000

_mesh = plsc.VectorSubcoreMesh(core_axis_name="c", subcore_axis_name="s")


def _mm_body(x_ref, win_ref, bin_ref, wg_ref, asrc_ref, adst_ref,
             h0_ref, h_ref, as_ref, ad_ref, mx_ref):
    x = x_ref[...]
    h0 = jnp.dot(x, win_ref[...], preferred_element_type=jnp.float32) + bin_ref[...]
    h = jnp.dot(h0, wg_ref[...], preferred_element_type=jnp.float32)
    h0_ref[...] = h0
    h_ref[...] = h
    a_s = jnp.dot(h, asrc_ref[...], preferred_element_type=jnp.float32)
    a_d = jnp.dot(h, adst_ref[...], preferred_element_type=jnp.float32)
    as_ref[...] = a_s
    ad_ref[...] = a_d
    ms = jnp.max(a_s)
    md = jnp.max(a_d)
    cur = jnp.concatenate([jnp.full((1, 128), ms, jnp.float32),
                           jnp.full((1, 128), md, jnp.float32)], axis=1)

    @pl.when(pl.program_id(0) == 0)
    def _():
        mx_ref[...] = cur

    @pl.when(pl.program_id(0) > 0)
    def _():
        mx_ref[...] = jnp.maximum(mx_ref[...], cur)


_RB = 2000


def _mm_call(x, w_in, b_in, w_g, att_src, att_dst):
    grid = (N // _RB,)
    return pl.pallas_call(
        _mm_body,
        grid=grid,
        in_specs=[
            pl.BlockSpec((_RB, D), lambda i: (i, 0)),
            pl.BlockSpec((D, H), lambda i: (0, 0)),
            pl.BlockSpec((1, H), lambda i: (0, 0)),
            pl.BlockSpec((H, H), lambda i: (0, 0)),
            pl.BlockSpec((H, 1), lambda i: (0, 0)),
            pl.BlockSpec((H, 1), lambda i: (0, 0)),
        ],
        out_specs=[
            pl.BlockSpec((_RB, H), lambda i: (i, 0)),
            pl.BlockSpec((_RB, H), lambda i: (i, 0)),
            pl.BlockSpec((_RB, 1), lambda i: (i, 0)),
            pl.BlockSpec((_RB, 1), lambda i: (i, 0)),
            pl.BlockSpec((1, 256), lambda i: (0, 0)),
        ],
        out_shape=[
            jax.ShapeDtypeStruct((N, H), jnp.float32),
            jax.ShapeDtypeStruct((N, H), jnp.float32),
            jax.ShapeDtypeStruct((N, 1), jnp.float32),
            jax.ShapeDtypeStruct((N, 1), jnp.float32),
            jax.ShapeDtypeStruct((1, 256), jnp.float32),
        ],
        compiler_params=pltpu.CompilerParams(
            dimension_semantics=("arbitrary",)),
    )(x, w_in, b_in, w_g, att_src, att_dst)


@functools.partial(
    pl.kernel,
    mesh=_mesh,
    out_type=jax.ShapeDtypeStruct((2, NDEN), jnp.float32),
    compiler_params=pltpu.CompilerParams(
        needs_layout_passes=False, use_tc_tiling_on_sc=False),
    scratch_types=[
        pltpu.VMEM((N,), jnp.float32),
        pltpu.VMEM((N,), jnp.float32),
        pltpu.VMEM((NB_A, KB), jnp.int32),
        pltpu.VMEM((NB_A, KB), jnp.int32),
        pltpu.VMEM((KB,), jnp.float32),
        pltpu.VMEM((16,), jnp.float32),
        pltpu.VMEM_SHARED((NDEN,), jnp.float32),
    ],
)
def _sc_den(as_hbm, ad_hbm, src_hbm, dst_hbm, shift_hbm, z640_hbm, denp_hbm,
            as_v, ad_v, src_v, dst_v, ex_v, shift_v, den_sh):
    c = lax.axis_index("c")
    s = lax.axis_index("s")
    wid = c * 16 + s
    off = pl.multiple_of(s * 640, 640)
    pltpu.sync_copy(z640_hbm, den_sh.at[pl.ds(off, 640)])
    pltpu.sync_copy(as_hbm, as_v)
    pltpu.sync_copy(ad_hbm, ad_v)
    pltpu.sync_copy(src_hbm.at[wid], src_v)
    pltpu.sync_copy(dst_hbm.at[wid], dst_v)
    pltpu.sync_copy(shift_hbm, shift_v)
    plsc.subcore_barrier()
    shift = shift_v[...]

    def body(b, carry):
        for cc in range(KB // 16):
            sv = src_v[b, pl.ds(cc * 16, 16)]
            dv = dst_v[b, pl.ds(cc * 16, 16)]
            ls = plsc.load_gather(as_v, [sv])
            ld = plsc.load_gather(ad_v, [dv])
            e = ls + ld
            e = jnp.where(e > 0.0, e, 0.2 * e)
            ex_v[pl.ds(cc * 16, 16)] = jnp.exp(e - shift)
        pltpu.sync_copy(ex_v, den_sh.at[dst_v.at[b]], add=True)
        return carry

    lax.fori_loop(0, NB_A, body, 0)
    plsc.subcore_barrier()
    pltpu.sync_copy(den_sh.at[pl.ds(off, 640)],
                    denp_hbm.at[c, pl.ds(off, 640)])


@functools.partial(
    pl.kernel,
    mesh=_mesh,
    out_type=jax.ShapeDtypeStruct((NGB, NAGG, HGB), jnp.float32),
    compiler_params=pltpu.CompilerParams(
        needs_layout_passes=False, use_tc_tiling_on_sc=False),
    scratch_types=[
        pltpu.VMEM((N,), jnp.float32),
        pltpu.VMEM((N,), jnp.float32),
        pltpu.VMEM((2, NDEN), jnp.float32),
        pltpu.VMEM((NDEN,), jnp.float32),
        pltpu.VMEM((NB_B, KBB), jnp.int32),
        pltpu.VMEM((NB_B, KBB), jnp.int32),
        pltpu.VMEM((NB_B * KBB,), jnp.float32),
        pltpu.VMEM((2, KBB, HGB), jnp.float32),
        pltpu.VMEM((2, KBB, HGB), jnp.float32),
        pltpu.VMEM((16,), jnp.float32),
        pltpu.VMEM_SHARED((NAGG, HGB), jnp.float32),
        pltpu.SemaphoreType.DMA,
        pltpu.SemaphoreType.DMA,
    ],
)
def _sc_agg(as_hbm, ad_hbm, denp_hbm, src_hbm, dst_hbm, shift_hbm,
            zrows_hbm, hq_hbm, aggq_hbm,
            as_v, ad_v, denp_v, den_v, src_v, dst_v, alpha_v, grows_v,
            srows_v, shift_v, agg_sh, gsem, ssem):
    c = lax.axis_index("c")
    s = lax.axis_index("s")
    roff = pl.multiple_of(s * 624, 8)
    last = s == 15
    pltpu.sync_copy(as_hbm, as_v)
    pltpu.sync_copy(ad_hbm, ad_v)
    pltpu.sync_copy(denp_hbm, denp_v)
    pltpu.sync_copy(src_hbm.at[s], src_v)
    pltpu.sync_copy(dst_hbm.at[s], dst_v)
    pltpu.sync_copy(shift_hbm, shift_v)

    def dsum(i, carry):
        o = pl.multiple_of(i * 16, 16)
        den_v[pl.ds(o, 16)] = denp_v[0, pl.ds(o, 16)] + denp_v[1, pl.ds(o, 16)]
        return carry

    lax.fori_loop(0, NDEN // 16, dsum, 0)
    shift = shift_v[...]
    nofs = jnp.full((16,), N, jnp.int32)
    ofs0 = jnp.full((16,), 4 * N, jnp.int32) * c

    for q in range(4):
        @pl.when(jnp.logical_not(last))
        def _(q=q):
            pltpu.sync_copy(zrows_hbm.at[pl.ds(0, 624)],
                            agg_sh.at[pl.ds(roff, 624)])

        @pl.when(last)
        def _(q=q):
            pltpu.sync_copy(zrows_hbm, agg_sh.at[pl.ds(9360, 640)])

        def prep(b, carry, q=q):
            for cc in range(KBB // 16):
                sl = pl.ds(cc * 16, 16)
                if q == 0:
                    sv = src_v[b, sl]
                    dv = dst_v[b, sl]
                    ls = plsc.load_gather(as_v, [sv])
                    ld = plsc.load_gather(ad_v, [dv])
                    e = ls + ld
                    e = jnp.where(e > 0.0, e, 0.2 * e)
                    ex = jnp.exp(e - shift)
                    dd = plsc.load_gather(den_v, [dv])
                    alpha_v[pl.ds(pl.multiple_of(b * KBB, 16) + cc * 16, 16)] = ex / dd
                    src_v[b, sl] = sv + ofs0
                else:
                    src_v[b, sl] = src_v[b, sl] + nofs
            return carry

        lax.fori_loop(0, NB_B, prep, 0)
        plsc.subcore_barrier()
        pltpu.async_copy(hq_hbm.at[src_v.at[0]], grows_v.at[0], gsem)

        def body(b, carry):
            bi = lax.rem(b, 2)
            ni = lax.rem(b + 1, 2)

            @pl.when(b >= 1)
            def _():
                pltpu.make_async_copy(
                    srows_v.at[0], agg_sh.at[dst_v.at[0]], ssem).wait()

            @pl.when(b + 1 < NB_B)
            def _():
                pltpu.async_copy(
                    hq_hbm.at[src_v.at[b + 1]], grows_v.at[ni], gsem)

            pltpu.make_async_copy(
                hq_hbm.at[src_v.at[b]], grows_v.at[bi], gsem).wait()

            abase = jnp.full((16,), b * KBB, jnp.int32)

            def scale(k16, carry2):
                for dk in range(16):
                    k = k16 * 16 + dk
                    av = plsc.load_gather(alpha_v, [abase + k])
                    srows_v[bi, k, pl.ds(0, HGB)] = (
                        grows_v[bi, k, pl.ds(0, HGB)] * av)
                return carry2

            lax.fori_loop(0, KBB // 16, scale, 0)
            pltpu.async_copy(srows_v.at[bi], agg_sh.at[dst_v.at[b]], ssem,
                             add=True)
            return carry

        lax.fori_loop(0, NB_B, body, 0)
        pltpu.make_async_copy(
            srows_v.at[0], agg_sh.at[dst_v.at[0]], ssem).wait()
        plsc.subcore_barrier()

        @pl.when(jnp.logical_not(last))
        def _(q=q):
            pltpu.sync_copy(agg_sh.at[pl.ds(roff, 624)],
                            aggq_hbm.at[4 * c + q, pl.ds(roff, 624)])

        @pl.when(last)
        def _(q=q):
            pltpu.sync_copy(agg_sh.at[pl.ds(9360, 640)],
                            aggq_hbm.at[4 * c + q, pl.ds(9360, 640)])


def _out_body(h0_ref, aggq_ref, bias_ref, wout_ref, bout_ref, o_ref):
    agg = jnp.concatenate(
        [aggq_ref[g] for g in range(NGB)], axis=1) + bias_ref[...]
    h1 = jnp.maximum(agg, 0.0)
    h2 = h0_ref[...] + h1
    lg = jnp.dot(h2, wout_ref[...], preferred_element_type=jnp.float32) + bout_ref[...]
    m = jnp.max(lg, axis=1, keepdims=True)
    ex = jnp.exp(lg - m)
    lse = jnp.log(jnp.sum(ex, axis=1, keepdims=True))
    o_ref[...] = lg - m - lse


def _out_call(h0, aggq, bias_g, w_out, b_out):
    grid = (N // _RB,)
    return pl.pallas_call(
        _out_body,
        grid=grid,
        in_specs=[
            pl.BlockSpec((_RB, H), lambda i: (i, 0)),
            pl.BlockSpec((NGB, _RB, HGB), lambda i: (0, i, 0)),
            pl.BlockSpec((1, H), lambda i: (0, 0)),
            pl.BlockSpec((H, OUT), lambda i: (0, 0)),
            pl.BlockSpec((1, OUT), lambda i: (0, 0)),
        ],
        out_specs=pl.BlockSpec((_RB, OUT), lambda i: (i, 0)),
        out_shape=jax.ShapeDtypeStruct((N, OUT), jnp.float32),
    )(h0, aggq, bias_g, w_out, b_out)


def kernel(x, edge_index, W_in, b_in, W_g, att_src, att_dst, bias_g, W_out, b_out):
    src_a = edge_index[0].reshape(32, NB_A, KB)
    dst_a = edge_index[1].reshape(32, NB_A, KB)
    src_b = edge_index[0].reshape(16, NB_B, KBB)
    dst_b = edge_index[1].reshape(16, NB_B, KBB)
    h0, h, a_s, a_d, mx = _mm_call(
        x, W_in, b_in.reshape(1, H), W_g,
        att_src.reshape(H, 1), att_dst.reshape(H, 1))
    a_s = a_s.reshape(N)
    a_d = a_d.reshape(N)
    hq = h.reshape(N, NGB, HGB).transpose(1, 0, 2).reshape(NGB * N, HGB)
    shift = jnp.maximum(mx[0, 0] + mx[0, 128], 0.0)
    shift_v = jnp.full((16,), shift, jnp.float32)
    z640 = jnp.zeros((640,), jnp.float32)
    zrows = jnp.zeros((640, HGB), jnp.float32)
    denp = _sc_den(a_s, a_d, src_a, dst_a, shift_v, z640)
    aggq = _sc_agg(a_s, a_d, denp, src_b, dst_b, shift_v, zrows, hq)
    return _out_call(h0, aggq, bias_g.reshape(1, H), W_out, b_out.reshape(1, OUT))


# speedup vs baseline: 1.0478x; 1.0324x over previous
import functools

import jax
import jax.numpy as jnp
from jax import lax
from jax.experimental import pallas as pl
from jax.experimental.pallas import tpu as pltpu
from jax.experimental.pallas import tpu_sc as plsc

N = 10000
E = 320000
D = 128
H = 128
HGB = 16
NGB = H // HGB
OUT = 16

KB = 80
KBB = 160
NB_A = E // 32 // KB
NB_B = E // 16 // KBB
NDEN = 10240
NAGG = 10000

_mesh = plsc.VectorSubcoreMesh(core_axis_name="c", subcore_axis_name="s")


def _mm_body(x_ref, win_ref, bin_ref, wg_ref, asrc_ref, adst_ref,
             h0_ref, h_ref, as_ref, ad_ref, mx_ref):
    x = x_ref[...]
    h0 = jnp.dot(x, win_ref[...], preferred_element_type=jnp.float32) + bin_ref[...]
    h = jnp.dot(h0, wg_ref[...], preferred_element_type=jnp.float32)
    h0_ref[...] = h0
    h_ref[...] = h
    a_s = jnp.dot(h, asrc_ref[...], preferred_element_type=jnp.float32)
    a_d = jnp.dot(h, adst_ref[...], preferred_element_type=jnp.float32)
    as_ref[...] = a_s
    ad_ref[...] = a_d
    ms = jnp.max(a_s)
    md = jnp.max(a_d)
    cur = jnp.concatenate([jnp.full((1, 128), ms, jnp.float32),
                           jnp.full((1, 128), md, jnp.float32)], axis=1)

    @pl.when(pl.program_id(0) == 0)
    def _():
        mx_ref[...] = cur

    @pl.when(pl.program_id(0) > 0)
    def _():
        mx_ref[...] = jnp.maximum(mx_ref[...], cur)


_RB = 2000


def _mm_call(x, w_in, b_in, w_g, att_src, att_dst):
    grid = (N // _RB,)
    return pl.pallas_call(
        _mm_body,
        grid=grid,
        in_specs=[
            pl.BlockSpec((_RB, D), lambda i: (i, 0)),
            pl.BlockSpec((D, H), lambda i: (0, 0)),
            pl.BlockSpec((1, H), lambda i: (0, 0)),
            pl.BlockSpec((H, H), lambda i: (0, 0)),
            pl.BlockSpec((H, 1), lambda i: (0, 0)),
            pl.BlockSpec((H, 1), lambda i: (0, 0)),
        ],
        out_specs=[
            pl.BlockSpec((_RB, H), lambda i: (i, 0)),
            pl.BlockSpec((_RB, H), lambda i: (i, 0)),
            pl.BlockSpec((_RB, 1), lambda i: (i, 0)),
            pl.BlockSpec((_RB, 1), lambda i: (i, 0)),
            pl.BlockSpec((1, 256), lambda i: (0, 0)),
        ],
        out_shape=[
            jax.ShapeDtypeStruct((N, H), jnp.float32),
            jax.ShapeDtypeStruct((N, H), jnp.float32),
            jax.ShapeDtypeStruct((N, 1), jnp.float32),
            jax.ShapeDtypeStruct((N, 1), jnp.float32),
            jax.ShapeDtypeStruct((1, 256), jnp.float32),
        ],
        compiler_params=pltpu.CompilerParams(
            dimension_semantics=("arbitrary",)),
    )(x, w_in, b_in, w_g, att_src, att_dst)


@functools.partial(
    pl.kernel,
    mesh=_mesh,
    out_type=jax.ShapeDtypeStruct((2, NDEN), jnp.float32),
    compiler_params=pltpu.CompilerParams(
        needs_layout_passes=False, use_tc_tiling_on_sc=False),
    scratch_types=[
        pltpu.VMEM((N,), jnp.float32),
        pltpu.VMEM((N,), jnp.float32),
        pltpu.VMEM((NB_A, KB), jnp.int32),
        pltpu.VMEM((NB_A, KB), jnp.int32),
        pltpu.VMEM((KB,), jnp.float32),
        pltpu.VMEM((16,), jnp.float32),
        pltpu.VMEM_SHARED((NDEN,), jnp.float32),
    ],
)
def _sc_den(as_hbm, ad_hbm, src_hbm, dst_hbm, shift_hbm, z640_hbm, denp_hbm,
            as_v, ad_v, src_v, dst_v, ex_v, shift_v, den_sh):
    c = lax.axis_index("c")
    s = lax.axis_index("s")
    wid = c * 16 + s
    off = pl.multiple_of(s * 640, 640)
    pltpu.sync_copy(z640_hbm, den_sh.at[pl.ds(off, 640)])
    pltpu.sync_copy(as_hbm, as_v)
    pltpu.sync_copy(ad_hbm, ad_v)
    pltpu.sync_copy(src_hbm.at[wid], src_v)
    pltpu.sync_copy(dst_hbm.at[wid], dst_v)
    pltpu.sync_copy(shift_hbm, shift_v)
    plsc.subcore_barrier()
    shift = shift_v[...]

    def body(b, carry):
        for cc in range(KB // 16):
            sv = src_v[b, pl.ds(cc * 16, 16)]
            dv = dst_v[b, pl.ds(cc * 16, 16)]
            ls = plsc.load_gather(as_v, [sv])
            ld = plsc.load_gather(ad_v, [dv])
            e = ls + ld
            e = jnp.where(e > 0.0, e, 0.2 * e)
            ex_v[pl.ds(cc * 16, 16)] = jnp.exp(e - shift)
        pltpu.sync_copy(ex_v, den_sh.at[dst_v.at[b]], add=True)
        return carry

    lax.fori_loop(0, NB_A, body, 0)
    plsc.subcore_barrier()
    pltpu.sync_copy(den_sh.at[pl.ds(off, 640)],
                    denp_hbm.at[c, pl.ds(off, 640)])


@functools.partial(
    pl.kernel,
    mesh=_mesh,
    out_type=jax.ShapeDtypeStruct((NGB, NAGG, HGB), jnp.float32),
    compiler_params=pltpu.CompilerParams(
        needs_layout_passes=False, use_tc_tiling_on_sc=False),
    scratch_types=[
        pltpu.VMEM((N,), jnp.float32),
        pltpu.VMEM((N,), jnp.float32),
        pltpu.VMEM((2, NDEN), jnp.float32),
        pltpu.VMEM((NDEN,), jnp.float32),
        pltpu.VMEM((NB_B, KBB), jnp.int32),
        pltpu.VMEM((NB_B, KBB), jnp.int32),
        pltpu.VMEM((NB_B * KBB,), jnp.float32),
        pltpu.VMEM((2, KBB, HGB), jnp.float32),
        pltpu.VMEM((2, KBB, HGB), jnp.float32),
        pltpu.VMEM((16,), jnp.float32),
        pltpu.VMEM_SHARED((NAGG, HGB), jnp.float32),
        pltpu.SemaphoreType.DMA,
        pltpu.SemaphoreType.DMA,
    ],
)
def _sc_agg(as_hbm, ad_hbm, denp_hbm, src_hbm, dst_hbm, shift_hbm,
            zrows_hbm, hq_hbm, aggq_hbm,
            as_v, ad_v, denp_v, den_v, src_v, dst_v, alpha_v, grows_v,
            srows_v, shift_v, agg_sh, gsem, ssem):
    c = lax.axis_index("c")
    s = lax.axis_index("s")
    roff = pl.multiple_of(s * 624, 8)
    last = s == 15
    pltpu.sync_copy(as_hbm, as_v)
    pltpu.sync_copy(ad_hbm, ad_v)
    pltpu.sync_copy(denp_hbm, denp_v)
    pltpu.sync_copy(src_hbm.at[s], src_v)
    pltpu.sync_copy(dst_hbm.at[s], dst_v)
    pltpu.sync_copy(shift_hbm, shift_v)

    def dsum(i, carry):
        o = pl.multiple_of(i * 16, 16)
        den_v[pl.ds(o, 16)] = denp_v[0, pl.ds(o, 16)] + denp_v[1, pl.ds(o, 16)]
        return carry

    lax.fori_loop(0, NDEN // 16, dsum, 0)
    shift = shift_v[...]
    nofs = jnp.full((16,), N, jnp.int32)
    ofs0 = jnp.full((16,), 4 * N, jnp.int32) * c

    for q in range(4):
        @pl.when(jnp.logical_not(last))
        def _(q=q):
            pltpu.sync_copy(zrows_hbm.at[pl.ds(0, 624)],
                            agg_sh.at[pl.ds(roff, 624)])

        @pl.when(last)
        def _(q=q):
            pltpu.sync_copy(zrows_hbm, agg_sh.at[pl.ds(9360, 640)])

        def prep_one(b, q=q):
            for cc in range(KBB // 16):
                sl = pl.ds(cc * 16, 16)
                if q == 0:
                    sv = src_v[b, sl]
                    dv = dst_v[b, sl]
                    ls = plsc.load_gather(as_v, [sv])
                    ld = plsc.load_gather(ad_v, [dv])
                    e = ls + ld
                    e = jnp.where(e > 0.0, e, 0.2 * e)
                    ex = jnp.exp(e - shift)
                    dd = plsc.load_gather(den_v, [dv])
                    alpha_v[pl.ds(pl.multiple_of(b * KBB, 16) + cc * 16, 16)] = ex / dd
                    src_v[b, sl] = sv + ofs0
                else:
                    src_v[b, sl] = src_v[b, sl] + nofs

        plsc.subcore_barrier()
        prep_one(0)
        pltpu.async_copy(hq_hbm.at[src_v.at[0]], grows_v.at[0], gsem)

        def body(b, carry, prep_one=prep_one):
            bi = lax.rem(b, 2)
            ni = lax.rem(b + 1, 2)

            @pl.when(b + 1 < NB_B)
            def _():
                prep_one(b + 1)
                pltpu.async_copy(
                    hq_hbm.at[src_v.at[b + 1]], grows_v.at[ni], gsem)

            @pl.when(b >= 1)
            def _():
                pltpu.make_async_copy(
                    srows_v.at[0], agg_sh.at[dst_v.at[0]], ssem).wait()

            pltpu.make_async_copy(
                hq_hbm.at[src_v.at[b]], grows_v.at[bi], gsem).wait()

            abase = jnp.full((16,), b * KBB, jnp.int32)

            def scale(k16, carry2):
                for dk in range(16):
                    k = k16 * 16 + dk
                    av = plsc.load_gather(alpha_v, [abase + k])
                    srows_v[bi, k, pl.ds(0, HGB)] = (
                        grows_v[bi, k, pl.ds(0, HGB)] * av)
                return carry2

            lax.fori_loop(0, KBB // 16, scale, 0)
            pltpu.async_copy(srows_v.at[bi], agg_sh.at[dst_v.at[b]], ssem,
                             add=True)
            return carry

        lax.fori_loop(0, NB_B, body, 0)
        pltpu.make_async_copy(
            srows_v.at[0], agg_sh.at[dst_v.at[0]], ssem).wait()
        plsc.subcore_barrier()

        @pl.when(jnp.logical_not(last))
        def _(q=q):
            pltpu.sync_copy(agg_sh.at[pl.ds(roff, 624)],
                            aggq_hbm.at[4 * c + q, pl.ds(roff, 624)])

        @pl.when(last)
        def _(q=q):
            pltpu.sync_copy(agg_sh.at[pl.ds(9360, 640)],
                            aggq_hbm.at[4 * c + q, pl.ds(9360, 640)])


def _out_body(h0_ref, aggq_ref, bias_ref, wout_ref, bout_ref, o_ref):
    agg = jnp.concatenate(
        [aggq_ref[g] for g in range(NGB)], axis=1) + bias_ref[...]
    h1 = jnp.maximum(agg, 0.0)
    h2 = h0_ref[...] + h1
    lg = jnp.dot(h2, wout_ref[...], preferred_element_type=jnp.float32) + bout_ref[...]
    m = jnp.max(lg, axis=1, keepdims=True)
    ex = jnp.exp(lg - m)
    lse = jnp.log(jnp.sum(ex, axis=1, keepdims=True))
    o_ref[...] = lg - m - lse


def _out_call(h0, aggq, bias_g, w_out, b_out):
    grid = (N // _RB,)
    return pl.pallas_call(
        _out_body,
        grid=grid,
        in_specs=[
            pl.BlockSpec((_RB, H), lambda i: (i, 0)),
            pl.BlockSpec((NGB, _RB, HGB), lambda i: (0, i, 0)),
            pl.BlockSpec((1, H), lambda i: (0, 0)),
            pl.BlockSpec((H, OUT), lambda i: (0, 0)),
            pl.BlockSpec((1, OUT), lambda i: (0, 0)),
        ],
        out_specs=pl.BlockSpec((_RB, OUT), lambda i: (i, 0)),
        out_shape=jax.ShapeDtypeStruct((N, OUT), jnp.float32),
    )(h0, aggq, bias_g, w_out, b_out)


def kernel(x, edge_index, W_in, b_in, W_g, att_src, att_dst, bias_g, W_out, b_out):
    src_a = edge_index[0].reshape(32, NB_A, KB)
    dst_a = edge_index[1].reshape(32, NB_A, KB)
    src_b = edge_index[0].reshape(16, NB_B, KBB)
    dst_b = edge_index[1].reshape(16, NB_B, KBB)
    h0, h, a_s, a_d, mx = _mm_call(
        x, W_in, b_in.reshape(1, H), W_g,
        att_src.reshape(H, 1), att_dst.reshape(H, 1))
    a_s = a_s.reshape(N)
    a_d = a_d.reshape(N)
    hq = h.reshape(N, NGB, HGB).transpose(1, 0, 2).reshape(NGB * N, HGB)
    shift = jnp.maximum(mx[0, 0] + mx[0, 128], 0.0)
    shift_v = jnp.full((16,), shift, jnp.float32)
    z640 = jnp.zeros((640,), jnp.float32)
    zrows = jnp.zeros((640, HGB), jnp.float32)
    denp = _sc_den(a_s, a_d, src_a, dst_a, shift_v, z640)
    aggq = _sc_agg(a_s, a_d, denp, src_b, dst_b, shift_v, zrows, hq)
    return _out_call(h0, aggq, bias_g.reshape(1, H), W_out, b_out.reshape(1, OUT))
